# baseline (speedup 1.0000x reference)
import functools

import jax
import jax.numpy as jnp
from jax import lax
from jax.experimental import pallas as pl
from jax.experimental.pallas import tpu as pltpu
from jax.experimental.pallas import tpu_sc as plsc

_SEQ = 8192
_HID = 2048
_NC, _NS = 2, 16
_NW = _NC * _NS
_ROWS_PER_W = _SEQ // _NW
_CHUNK = 8
_NBUF = 6
_N_CHUNKS = _ROWS_PER_W // _CHUNK


def _sc_copy(table_hbm, out_hbm, *scratch):
    wid = lax.axis_index("s") * _NC + lax.axis_index("c")
    base = wid * _ROWS_PER_W
    bufs = scratch[:_NBUF]
    isems = scratch[_NBUF:2 * _NBUF]
    osems = scratch[2 * _NBUF:3 * _NBUF]

    def in_copy(c):
        return pltpu.make_async_copy(
            table_hbm.at[pl.ds(base + c * _CHUNK, _CHUNK), :],
            bufs[c % _NBUF], isems[c % _NBUF])

    def out_copy(c):
        return pltpu.make_async_copy(
            bufs[c % _NBUF],
            out_hbm.at[pl.ds(base + c * _CHUNK, _CHUNK), :], osems[c % _NBUF])

    k = _NBUF // 2
    for c in range(k):
        in_copy(c).start()
    for c in range(_N_CHUNKS):
        in_copy(c).wait()
        out_copy(c).start()
        nxt = c + k
        if nxt < _N_CHUNKS:
            prev = nxt - _NBUF
            if prev >= 0:
                out_copy(prev).wait()
            in_copy(nxt).start()
    for c in range(_N_CHUNKS - _NBUF, _N_CHUNKS):
        out_copy(c).wait()


_sc_kernel = functools.partial(
    pl.kernel,
    out_type=jax.ShapeDtypeStruct((_SEQ, _HID), jnp.float32),
    mesh=plsc.VectorSubcoreMesh(core_axis_name="c", subcore_axis_name="s"),
    scratch_types=(
        [pltpu.VMEM((_CHUNK, _HID), jnp.float32)] * _NBUF
        + [pltpu.SemaphoreType.DMA] * (2 * _NBUF)
    ),
)(_sc_copy)


def kernel(input_ids, pos_table):
    del input_ids
    out = _sc_kernel(pos_table)
    return out.reshape(1, _SEQ, _HID)


# speedup vs baseline: 1.0065x; 1.0065x over previous
import functools

import jax
import jax.numpy as jnp
from jax import lax
from jax.experimental import pallas as pl
from jax.experimental.pallas import tpu as pltpu
from jax.experimental.pallas import tpu_sc as plsc

_SEQ = 8192
_HID = 2048
_NC, _NS = 2, 16
_NW = _NC * _NS
_ROWS_PER_W = _SEQ // _NW
_CHUNK = 16
_N_CHUNKS = _ROWS_PER_W // _CHUNK


def _sc_copy(table_hbm, out_hbm, buf0, buf1, isem0, isem1, osem0, osem1):
    wid = lax.axis_index("s") * _NC + lax.axis_index("c")
    base = wid * _ROWS_PER_W
    bufs, isems, osems = (buf0, buf1), (isem0, isem1), (osem0, osem1)

    def in_copy(c, b):
        return pltpu.make_async_copy(
            table_hbm.at[pl.ds(base + c * _CHUNK, _CHUNK), :], bufs[b], isems[b])

    def out_copy(c, b):
        return pltpu.make_async_copy(
            bufs[b], out_hbm.at[pl.ds(base + c * _CHUNK, _CHUNK), :], osems[b])

    in_copy(0, 0).start()
    for c in range(_N_CHUNKS):
        b = c % 2
        nb = (c + 1) % 2
        if c + 1 < _N_CHUNKS:
            if c >= 1:
                out_copy(c - 1, nb).wait()
            in_copy(c + 1, nb).start()
        in_copy(c, b).wait()
        out_copy(c, b).start()
    out_copy(_N_CHUNKS - 2, (_N_CHUNKS - 2) % 2).wait()
    out_copy(_N_CHUNKS - 1, (_N_CHUNKS - 1) % 2).wait()


_sc_kernel = functools.partial(
    pl.kernel,
    out_type=jax.ShapeDtypeStruct((_SEQ, _HID), jnp.float32),
    mesh=plsc.VectorSubcoreMesh(core_axis_name="c", subcore_axis_name="s"),
    scratch_types=[
        pltpu.VMEM((_CHUNK, _HID), jnp.float32),
        pltpu.VMEM((_CHUNK, _HID), jnp.float32),
        pltpu.SemaphoreType.DMA,
        pltpu.SemaphoreType.DMA,
        pltpu.SemaphoreType.DMA,
        pltpu.SemaphoreType.DMA,
    ],
)(_sc_copy)


def kernel(input_ids, pos_table):
    del input_ids
    out = _sc_kernel(pos_table)
    return out.reshape(1, _SEQ, _HID)


# speedup vs baseline: 1.0141x; 1.0075x over previous
import functools

import jax
import jax.numpy as jnp
from jax import lax
from jax.experimental import pallas as pl
from jax.experimental.pallas import tpu as pltpu
from jax.experimental.pallas import tpu_sc as plsc

_SEQ = 8192
_HID = 2048
_NC, _NS = 2, 16
_NW = _NC * _NS
_ROWS_PER_W = _SEQ // _NW
_CHUNK = 16
_N_CHUNKS = _ROWS_PER_W // _CHUNK


def _sc_copy(table_hbm, out_hbm, buf0, buf1, isem0, isem1, osem0, osem1):
    wid = lax.axis_index("s") * _NC + lax.axis_index("c")
    base = wid * _ROWS_PER_W
    bufs, isems, osems = (buf0, buf1), (isem0, isem1), (osem0, osem1)

    def in_copy(c, b):
        return pltpu.make_async_copy(
            table_hbm.at[pl.ds(base + c * _CHUNK, _CHUNK), :], bufs[b], isems[b])

    def out_copy(c, b):
        return pltpu.make_async_copy(
            bufs[b], out_hbm.at[pl.ds(base + c * _CHUNK, _CHUNK), :], osems[b])

    in_copy(0, 0).start()

    def step(i, carry):
        c = i * 2
        @pl.when(i > 0)
        def _():
            out_copy(c - 1, 1).wait()

        in_copy(c + 1, 1).start()
        in_copy(c, 0).wait()
        out_copy(c, 0).start()

        @pl.when(i < _N_CHUNKS // 2 - 1)
        def _():
            out_copy(c, 0).wait()
            in_copy(c + 2, 0).start()

        in_copy(c + 1, 1).wait()
        out_copy(c + 1, 1).start()
        return carry

    lax.fori_loop(0, _N_CHUNKS // 2, step, 0)
    out_copy(_N_CHUNKS - 2, 0).wait()
    out_copy(_N_CHUNKS - 1, 1).wait()


_sc_kernel = functools.partial(
    pl.kernel,
    out_type=jax.ShapeDtypeStruct((_SEQ, _HID), jnp.float32),
    mesh=plsc.VectorSubcoreMesh(core_axis_name="c", subcore_axis_name="s"),
    scratch_types=[
        pltpu.VMEM((_CHUNK, _HID), jnp.float32),
        pltpu.VMEM((_CHUNK, _HID), jnp.float32),
        pltpu.SemaphoreType.DMA,
        pltpu.SemaphoreType.DMA,
        pltpu.SemaphoreType.DMA,
        pltpu.SemaphoreType.DMA,
    ],
)(_sc_copy)


def kernel(input_ids, pos_table):
    del input_ids
    out = _sc_kernel(pos_table)
    return out.reshape(1, _SEQ, _HID)
